# Initial kernel scaffold; baseline (speedup 1.0000x reference)
import jax
import jax.numpy as jnp
from jax.experimental import pallas as pl


def kernel(all_nodes, edge_index_dd, edge_index_dt, edge_index_tt, edge_attr_dd, W_dd, a_src_dd, a_dst_dd, a_edge_dd, W_dt, a_src_dt, a_dst_dt, W_tt, a_src_tt, a_dst_tt):
    raise NotImplementedError("write your pallas kernel here")


# speedup vs baseline: 20.2095x; 20.2095x over previous
import functools

import jax
import jax.numpy as jnp
from jax import lax
from jax.experimental import pallas as pl
from jax.experimental.pallas import tpu as pltpu
from jax.experimental.pallas import tpu_sc as plsc

N = 10000
D = 128
E = 320000
DE = 16
NPAD = 10240

NC = 2
NS = 16
NW = NC * NS
EPW = E // NW
CB = 80
NCH = EPW // CB
LPC = CB // 16
RPT = NPAD // NS

_mesh = plsc.VectorSubcoreMesh(core_axis_name="c", subcore_axis_name="s")


def _proj_body(x_ref, wdd_ref, wdt_ref, wtt_ref,
               asdd_ref, atdd_ref, asdt_ref, atdt_ref, astt_ref, attt_ref,
               hdd_ref, hdt_ref, htt_ref,
               sdd_ref, tdd_ref, sdt_ref, tdt_ref, stt_ref, ttt_ref):
    x = x_ref[...]
    wrefs = (wdd_ref, wdt_ref, wtt_ref)
    arefs = (asdd_ref, atdd_ref, asdt_ref, atdt_ref, astt_ref, attt_ref)
    hrefs = (hdd_ref, hdt_ref, htt_ref)
    srefs = (sdd_ref, tdd_ref, sdt_ref, tdt_ref, stt_ref, ttt_ref)
    for r in range(3):
        h = jnp.dot(x, wrefs[r][...], preferred_element_type=jnp.float32)
        hrefs[r][...] = h
        h3 = h.reshape(8, 128, D)
        for p in range(2):
            a = arefs[2 * r + p][0, :]
            srefs[2 * r + p][...] = jnp.sum(h3 * a[None, None, :], axis=2)


def _project(x, W_dd, W_dt, W_tt, a6):
    blk = 1024
    grid = (NPAD // blk,)
    wspec = pl.BlockSpec((D, D), lambda i: (0, 0))
    aspec = pl.BlockSpec((1, D), lambda i: (0, 0))
    hspec = pl.BlockSpec((blk, D), lambda i: (i, 0))
    sspec = pl.BlockSpec((8, D), lambda i: (i, 0))
    return pl.pallas_call(
        _proj_body,
        grid=grid,
        in_specs=[hspec, wspec, wspec, wspec] + [aspec] * 6,
        out_specs=[hspec] * 3 + [sspec] * 6,
        out_shape=[jax.ShapeDtypeStruct((NPAD, D), jnp.float32)] * 3
        + [jax.ShapeDtypeStruct((NPAD // 128, 128), jnp.float32)] * 6,
    )(x, W_dd, W_dt, W_tt, *a6)


def _edge_term_body(ea_ref, b_ref, o_ref):
    o_ref[...] = jnp.dot(ea_ref[...], b_ref[...],
                         preferred_element_type=jnp.float32)


def _edge_term(ea2, B):
    blk = 2000
    rows = ea2.shape[0]
    return pl.pallas_call(
        _edge_term_body,
        grid=(rows // blk,),
        in_specs=[pl.BlockSpec((blk, 128), lambda i: (i, 0)),
                  pl.BlockSpec((128, 128), lambda i: (0, 0))],
        out_specs=pl.BlockSpec((blk, 128), lambda i: (i, 0)),
        out_shape=jax.ShapeDtypeStruct((rows, 128), jnp.float32),
    )(ea2, B)


@functools.partial(
    pl.kernel,
    out_type=[jax.ShapeDtypeStruct((E,), jnp.float32)] * 3
    + [jax.ShapeDtypeStruct((NC, NPAD), jnp.float32)] * 3,
    mesh=_mesh,
    scratch_types=[
        pltpu.VMEM((NPAD,), jnp.float32),
        pltpu.VMEM((NPAD,), jnp.float32),
        pltpu.VMEM((EPW,), jnp.int32),
        pltpu.VMEM((NCH, CB), jnp.int32),
        pltpu.VMEM((EPW,), jnp.float32),
        pltpu.VMEM((EPW,), jnp.float32),
        pltpu.VMEM_SHARED((NPAD,), jnp.float32),
        pltpu.VMEM_SHARED((NPAD,), jnp.float32),
        pltpu.VMEM_SHARED((NPAD,), jnp.float32),
    ],
    compiler_params=pltpu.CompilerParams(needs_layout_passes=False),
)
def _sc_edge_logits(s_dd, t_dd, s_dt, t_dt, s_tt, t_tt,
                    src_dd, src_dt, src_tt, dst2_dd, dst2_dt, dst2_tt,
                    eatt_hbm,
                    ex_dd, ex_dt, ex_tt, den_dd, den_dt, den_tt,
                    s_v, t_v, src_v, dst_v, ea_v, ex_v, dsh0, dsh1, dsh2):
    cid = lax.axis_index("c")
    sid = lax.axis_index("s")
    wid = cid * NS + sid
    base = pl.multiple_of(wid * EPW, 8)
    rbase = pl.multiple_of(sid * RPT, 8)
    dshs = (dsh0, dsh1, dsh2)
    s_hbms = (s_dd, s_dt, s_tt)
    t_hbms = (t_dd, t_dt, t_tt)
    src_hbms = (src_dd, src_dt, src_tt)
    dst_hbms = (dst2_dd, dst2_dt, dst2_tt)
    ex_hbms = (ex_dd, ex_dt, ex_tt)
    den_hbms = (den_dd, den_dt, den_tt)

    for m in range(RPT // 16):
        ex_v[pl.ds(m * 16, 16)] = jnp.zeros((16,), jnp.float32)
    for r in range(3):
        pltpu.sync_copy(ex_v.at[pl.ds(0, RPT)], dshs[r].at[pl.ds(rbase, RPT)])
    plsc.subcore_barrier()

    for r in range(3):
        pltpu.sync_copy(s_hbms[r], s_v)
        pltpu.sync_copy(t_hbms[r], t_v)
        pltpu.sync_copy(src_hbms[r].at[pl.ds(base, EPW)], src_v)
        pltpu.sync_copy(dst_hbms[r].at[wid], dst_v)
        if r == 0:
            pltpu.sync_copy(eatt_hbm.at[pl.ds(base, EPW)], ea_v)

        def chunk_body(ch, carry, r=r):
            for k in range(LPC):
                off = ch * CB + k * 16
                si = src_v[pl.ds(off, 16)]
                ti = dst_v[ch, pl.ds(k * 16, 16)]
                e = plsc.load_gather(s_v, [si]) + plsc.load_gather(t_v, [ti])
                if r == 0:
                    e = e + ea_v[pl.ds(off, 16)]
                e = jnp.where(e >= 0.0, e, 0.2 * e)
                ex_v[pl.ds(off, 16)] = jnp.exp(e)
            pltpu.sync_copy(ex_v.at[pl.ds(ch * CB, CB)],
                            dshs[r].at[dst_v.at[ch]], add=True)
            return carry

        lax.fori_loop(0, NCH, chunk_body, 0)
        pltpu.sync_copy(ex_v, ex_hbms[r].at[pl.ds(base, EPW)])

    plsc.subcore_barrier()
    for r in range(3):
        pltpu.sync_copy(dshs[r].at[pl.ds(rbase, RPT)],
                        den_hbms[r].at[cid, pl.ds(rbase, RPT)])


def _den_combine_body(d0_ref, d1_ref, d2_ref, o0_ref, o1_ref, o2_ref):
    for d_ref, o_ref in ((d0_ref, o0_ref), (d1_ref, o1_ref), (d2_ref, o2_ref)):
        o_ref[...] = d_ref[0] + d_ref[1] + jnp.float32(1e-16)


def _den_combine(den3):
    ispec = pl.BlockSpec((NC, 8, 128), lambda i: (0, i, 0))
    ospec = pl.BlockSpec((8, 128), lambda i: (i, 0))
    return pl.pallas_call(
        _den_combine_body,
        grid=(NPAD // 1024,),
        in_specs=[ispec] * 3,
        out_specs=[ospec] * 3,
        out_shape=[jax.ShapeDtypeStruct((NPAD // 128, 128), jnp.float32)] * 3,
    )(*[d.reshape(NC, NPAD // 128, 128) for d in den3])


@functools.partial(
    pl.kernel,
    out_type=jax.ShapeDtypeStruct((NC, NPAD, D), jnp.float32),
    mesh=_mesh,
    scratch_types=[
        pltpu.VMEM((NCH, CB), jnp.int32),
        pltpu.VMEM((NCH, CB), jnp.int32),
        pltpu.VMEM((CB,), jnp.float32),
        pltpu.VMEM((CB,), jnp.float32),
        pltpu.VMEM((CB, D), jnp.float32),
        pltpu.SemaphoreType.DMA,
        pltpu.VMEM_SHARED((NPAD, D), jnp.float32),
    ],
    compiler_params=pltpu.CompilerParams(needs_layout_passes=False),
)
def _sc_aggregate(h_dd, h_dt, h_tt, ex_dd, ex_dt, ex_tt,
                  src2_dd, src2_dt, src2_tt, dst2_dd, dst2_dt, dst2_tt,
                  den_dd, den_dt, den_tt,
                  out_hbm,
                  src2_v, dst2_v, ex_c, den_c, rows_v, sem, accum):
    cid = lax.axis_index("c")
    sid = lax.axis_index("s")
    wid = cid * NS + sid
    base = pl.multiple_of(wid * EPW, 8)
    h_hbms = (h_dd, h_dt, h_tt)
    ex_hbms = (ex_dd, ex_dt, ex_tt)
    src_hbms = (src2_dd, src2_dt, src2_tt)
    dst_hbms = (dst2_dd, dst2_dt, dst2_tt)
    den_hbms = (den_dd, den_dt, den_tt)

    for i in range(CB):
        for u in range(D // 16):
            rows_v[i, pl.ds(u * 16, 16)] = jnp.zeros((16,), jnp.float32)
    for q in range(RPT // CB):
        st = pl.multiple_of(sid * RPT + q * CB, 8)
        pltpu.sync_copy(rows_v, accum.at[pl.ds(st, CB)])
    plsc.subcore_barrier()

    for r in range(3):
        pltpu.sync_copy(src_hbms[r].at[wid], src2_v)
        pltpu.sync_copy(dst_hbms[r].at[wid], dst2_v)

        def row_body(ch, c, r=r):
            cbase = pl.multiple_of(base + ch * CB, 8)
            pltpu.sync_copy(ex_hbms[r].at[pl.ds(cbase, CB)], ex_c)
            pltpu.async_copy(den_hbms[r].at[dst2_v.at[ch]], den_c, sem).wait()
            pltpu.async_copy(h_hbms[r].at[src2_v.at[ch]], rows_v, sem).wait()

            def scale_body(g, c2):
                sl16 = pl.ds(g * 16, 16)
                av16 = ex_c[sl16] / den_c[sl16]
                for j in range(16):
                    av = jnp.full((16,), av16[j], jnp.float32)
                    i = g * 16 + j
                    for u in range(D // 16):
                        sl = pl.ds(u * 16, 16)
                        rows_v[i, sl] = rows_v[i, sl] * av
                return c2

            lax.fori_loop(0, LPC, scale_body, 0)
            pltpu.sync_copy(rows_v, accum.at[dst2_v.at[ch]], add=True)
            return c

        lax.fori_loop(0, NCH, row_body, 0)

    plsc.subcore_barrier()
    for q in range(RPT // CB):
        st = pl.multiple_of(sid * RPT + q * CB, 8)
        pltpu.sync_copy(accum.at[pl.ds(st, CB)],
                        out_hbm.at[cid, pl.ds(st, CB)])


def _combine_body(p0_ref, p1_ref, o_ref):
    o_ref[...] = (p0_ref[...] + p1_ref[...]) * jnp.float32(1.0 / 3.0)


def _combine(p0, p1):
    blk = 1024
    spec = pl.BlockSpec((blk, D), lambda i: (i, 0))
    return pl.pallas_call(
        _combine_body,
        grid=(NPAD // blk,),
        in_specs=[spec, spec],
        out_specs=spec,
        out_shape=jax.ShapeDtypeStruct((NPAD, D), jnp.float32),
    )(p0, p1)


def kernel(all_nodes, edge_index_dd, edge_index_dt, edge_index_tt, edge_attr_dd,
           W_dd, a_src_dd, a_dst_dd, a_edge_dd,
           W_dt, a_src_dt, a_dst_dt,
           W_tt, a_src_tt, a_dst_tt):
    x = jnp.concatenate(
        [all_nodes, jnp.zeros((NPAD - N, D), jnp.float32)], axis=0)
    a6 = [a.reshape(1, D) for a in
          (a_src_dd, a_dst_dd, a_src_dt, a_dst_dt, a_src_tt, a_dst_tt)]
    (h_dd, h_dt, h_tt,
     s2_dd, t2_dd, s2_dt, t2_dt, s2_tt, t2_tt) = _project(
        x, W_dd, W_dt, W_tt, a6)
    s_dd, t_dd = s2_dd.reshape(NPAD), t2_dd.reshape(NPAD)
    s_dt, t_dt = s2_dt.reshape(NPAD), t2_dt.reshape(NPAD)
    s_tt, t_tt = s2_tt.reshape(NPAD), t2_tt.reshape(NPAD)

    ea2 = edge_attr_dd.reshape(E // 8, 8 * DE)
    rows = ((jnp.arange(8) * DE)[:, None] + jnp.arange(DE)[None, :]).reshape(-1)
    cols = jnp.repeat(jnp.arange(8), DE)
    B = jnp.zeros((8 * DE, 128), jnp.float32).at[rows, cols].set(
        jnp.tile(a_edge_dd, 8))
    eatt = _edge_term(ea2, B)[:, :8].reshape(E)

    src_dd, dst_dd = edge_index_dd[0], edge_index_dd[1]
    src_dt, dst_dt = edge_index_dt[0], edge_index_dt[1]
    src_tt, dst_tt = edge_index_tt[0], edge_index_tt[1]
    d2 = lambda a: a.reshape(NW, NCH, CB)

    (ex_dd, ex_dt, ex_tt, denp_dd, denp_dt, denp_tt) = _sc_edge_logits(
        s_dd, t_dd, s_dt, t_dt, s_tt, t_tt,
        src_dd, src_dt, src_tt, d2(dst_dd), d2(dst_dt), d2(dst_tt), eatt)

    den_dd, den_dt, den_tt = [d.reshape(NPAD) for d in
                              _den_combine((denp_dd, denp_dt, denp_tt))]

    out_parts = _sc_aggregate(
        h_dd, h_dt, h_tt, ex_dd, ex_dt, ex_tt,
        d2(src_dd), d2(src_dt), d2(src_tt),
        d2(dst_dd), d2(dst_dt), d2(dst_tt),
        den_dd, den_dt, den_tt)

    out = _combine(out_parts[0], out_parts[1])
    return out[:N]


# speedup vs baseline: 35.9857x; 1.7806x over previous
import functools

import jax
import jax.numpy as jnp
from jax import lax
from jax.experimental import pallas as pl
from jax.experimental.pallas import tpu as pltpu
from jax.experimental.pallas import tpu_sc as plsc

N = 10000
D = 128
E = 320000
DE = 16
NPAD = 10240

NC = 2
NS = 16
NW = NC * NS
EPW = E // NW
CB = 80
NCH = EPW // CB
LPC = CB // 16
RPT = NPAD // NS

_mesh = plsc.VectorSubcoreMesh(core_axis_name="c", subcore_axis_name="s")


def _proj_body(x_ref, wdd_ref, wdt_ref, wtt_ref,
               asdd_ref, atdd_ref, asdt_ref, atdt_ref, astt_ref, attt_ref,
               hdd_ref, hdt_ref, htt_ref,
               sdd_ref, tdd_ref, sdt_ref, tdt_ref, stt_ref, ttt_ref):
    x = x_ref[...]
    wrefs = (wdd_ref, wdt_ref, wtt_ref)
    arefs = (asdd_ref, atdd_ref, asdt_ref, atdt_ref, astt_ref, attt_ref)
    hrefs = (hdd_ref, hdt_ref, htt_ref)
    srefs = (sdd_ref, tdd_ref, sdt_ref, tdt_ref, stt_ref, ttt_ref)
    for r in range(3):
        h = jnp.dot(x, wrefs[r][...], preferred_element_type=jnp.float32)
        hrefs[r][...] = h
        h3 = h.reshape(8, 128, D)
        for p in range(2):
            a = arefs[2 * r + p][0, :]
            srefs[2 * r + p][...] = jnp.sum(h3 * a[None, None, :], axis=2)


def _project(x, W_dd, W_dt, W_tt, a6):
    blk = 1024
    grid = (NPAD // blk,)
    wspec = pl.BlockSpec((D, D), lambda i: (0, 0))
    aspec = pl.BlockSpec((1, D), lambda i: (0, 0))
    hspec = pl.BlockSpec((blk, D), lambda i: (i, 0))
    sspec = pl.BlockSpec((8, D), lambda i: (i, 0))
    return pl.pallas_call(
        _proj_body,
        grid=grid,
        in_specs=[hspec, wspec, wspec, wspec] + [aspec] * 6,
        out_specs=[hspec] * 3 + [sspec] * 6,
        out_shape=[jax.ShapeDtypeStruct((NPAD, D), jnp.float32)] * 3
        + [jax.ShapeDtypeStruct((NPAD // 128, 128), jnp.float32)] * 6,
    )(x, W_dd, W_dt, W_tt, *a6)


def _edge_term_body(ea_ref, b_ref, o_ref):
    o_ref[...] = jnp.dot(ea_ref[...], b_ref[...],
                         preferred_element_type=jnp.float32)


def _edge_term(ea2, B):
    blk = 2000
    rows = ea2.shape[0]
    return pl.pallas_call(
        _edge_term_body,
        grid=(rows // blk,),
        in_specs=[pl.BlockSpec((blk, 128), lambda i: (i, 0)),
                  pl.BlockSpec((128, 128), lambda i: (0, 0))],
        out_specs=pl.BlockSpec((blk, 128), lambda i: (i, 0)),
        out_shape=jax.ShapeDtypeStruct((rows, 128), jnp.float32),
    )(ea2, B)


@functools.partial(
    pl.kernel,
    out_type=[jax.ShapeDtypeStruct((E,), jnp.float32)] * 3
    + [jax.ShapeDtypeStruct((NC, NPAD), jnp.float32)] * 3,
    mesh=_mesh,
    scratch_types=[
        pltpu.VMEM((NPAD,), jnp.float32),
        pltpu.VMEM((NPAD,), jnp.float32),
        pltpu.VMEM((EPW,), jnp.int32),
        pltpu.VMEM((NCH, CB), jnp.int32),
        pltpu.VMEM((EPW,), jnp.float32),
        pltpu.VMEM((EPW,), jnp.float32),
        pltpu.VMEM_SHARED((NPAD,), jnp.float32),
        pltpu.VMEM_SHARED((NPAD,), jnp.float32),
        pltpu.VMEM_SHARED((NPAD,), jnp.float32),
    ],
    compiler_params=pltpu.CompilerParams(needs_layout_passes=False),
)
def _sc_edge_logits(s_dd, t_dd, s_dt, t_dt, s_tt, t_tt,
                    src_dd, src_dt, src_tt, dst2_dd, dst2_dt, dst2_tt,
                    eatt_hbm,
                    ex_dd, ex_dt, ex_tt, den_dd, den_dt, den_tt,
                    s_v, t_v, src_v, dst_v, ea_v, ex_v, dsh0, dsh1, dsh2):
    cid = lax.axis_index("c")
    sid = lax.axis_index("s")
    wid = cid * NS + sid
    base = pl.multiple_of(wid * EPW, 8)
    rbase = pl.multiple_of(sid * RPT, 8)
    dshs = (dsh0, dsh1, dsh2)
    s_hbms = (s_dd, s_dt, s_tt)
    t_hbms = (t_dd, t_dt, t_tt)
    src_hbms = (src_dd, src_dt, src_tt)
    dst_hbms = (dst2_dd, dst2_dt, dst2_tt)
    ex_hbms = (ex_dd, ex_dt, ex_tt)
    den_hbms = (den_dd, den_dt, den_tt)

    for m in range(RPT // 16):
        ex_v[pl.ds(m * 16, 16)] = jnp.zeros((16,), jnp.float32)
    for r in range(3):
        pltpu.sync_copy(ex_v.at[pl.ds(0, RPT)], dshs[r].at[pl.ds(rbase, RPT)])
    plsc.subcore_barrier()

    for r in range(3):
        pltpu.sync_copy(s_hbms[r], s_v)
        pltpu.sync_copy(t_hbms[r], t_v)
        pltpu.sync_copy(src_hbms[r].at[pl.ds(base, EPW)], src_v)
        pltpu.sync_copy(dst_hbms[r].at[wid], dst_v)
        if r == 0:
            pltpu.sync_copy(eatt_hbm.at[pl.ds(base, EPW)], ea_v)

        def chunk_body(ch, carry, r=r):
            for k in range(LPC):
                off = ch * CB + k * 16
                si = src_v[pl.ds(off, 16)]
                ti = dst_v[ch, pl.ds(k * 16, 16)]
                e = plsc.load_gather(s_v, [si]) + plsc.load_gather(t_v, [ti])
                if r == 0:
                    e = e + ea_v[pl.ds(off, 16)]
                e = jnp.where(e >= 0.0, e, 0.2 * e)
                ex_v[pl.ds(off, 16)] = jnp.exp(e)
            pltpu.sync_copy(ex_v.at[pl.ds(ch * CB, CB)],
                            dshs[r].at[dst_v.at[ch]], add=True)
            return carry

        lax.fori_loop(0, NCH, chunk_body, 0)
        pltpu.sync_copy(ex_v, ex_hbms[r].at[pl.ds(base, EPW)])

    plsc.subcore_barrier()
    for r in range(3):
        pltpu.sync_copy(dshs[r].at[pl.ds(rbase, RPT)],
                        den_hbms[r].at[cid, pl.ds(rbase, RPT)])


def _den_combine_body(d0_ref, d1_ref, d2_ref, o0_ref, o1_ref, o2_ref):
    for d_ref, o_ref in ((d0_ref, o0_ref), (d1_ref, o1_ref), (d2_ref, o2_ref)):
        o_ref[...] = d_ref[0] + d_ref[1] + jnp.float32(1e-16)


def _den_combine(den3):
    ispec = pl.BlockSpec((NC, 8, 128), lambda i: (0, i, 0))
    ospec = pl.BlockSpec((8, 128), lambda i: (i, 0))
    return pl.pallas_call(
        _den_combine_body,
        grid=(NPAD // 1024,),
        in_specs=[ispec] * 3,
        out_specs=[ospec] * 3,
        out_shape=[jax.ShapeDtypeStruct((NPAD // 128, 128), jnp.float32)] * 3,
    )(*[d.reshape(NC, NPAD // 128, 128) for d in den3])


@functools.partial(
    pl.kernel,
    out_type=[jax.ShapeDtypeStruct((E,), jnp.float32)] * 3,
    mesh=_mesh,
    scratch_types=[
        pltpu.VMEM((EPW,), jnp.float32),
        pltpu.VMEM((NCH, CB), jnp.int32),
        pltpu.VMEM((EPW,), jnp.float32),
    ],
    compiler_params=pltpu.CompilerParams(needs_layout_passes=False),
)
def _sc_attn(ex_dd, ex_dt, ex_tt, dst2_dd, dst2_dt, dst2_tt,
             den_dd, den_dt, den_tt,
             att_dd, att_dt, att_tt,
             den_v, dst2_v, ex_v):
    cid = lax.axis_index("c")
    sid = lax.axis_index("s")
    wid = cid * NS + sid
    base = pl.multiple_of(wid * EPW, 8)
    ex_hbms = (ex_dd, ex_dt, ex_tt)
    dst_hbms = (dst2_dd, dst2_dt, dst2_tt)
    den_hbms = (den_dd, den_dt, den_tt)
    att_hbms = (att_dd, att_dt, att_tt)

    for r in range(3):
        pltpu.sync_copy(den_hbms[r].at[pl.ds(0, EPW)], den_v)
        pltpu.sync_copy(dst_hbms[r].at[wid], dst2_v)
        pltpu.sync_copy(ex_hbms[r].at[pl.ds(base, EPW)], ex_v)

        def chunk_body(ch, c):
            for k in range(LPC):
                sl = pl.ds(ch * CB + k * 16, 16)
                ti = dst2_v[ch, pl.ds(k * 16, 16)]
                ex_v[sl] = ex_v[sl] / plsc.load_gather(den_v, [ti])
            return c

        lax.fori_loop(0, NCH, chunk_body, 0)
        pltpu.sync_copy(ex_v, att_hbms[r].at[pl.ds(base, EPW)])


@functools.partial(
    pl.kernel,
    out_type=jax.ShapeDtypeStruct((NC, NPAD, D), jnp.float32),
    mesh=_mesh,
    scratch_types=[
        pltpu.VMEM((NCH, CB), jnp.int32),
        pltpu.VMEM((2, CB), jnp.int32),
        pltpu.VMEM((2, CB), jnp.float32),
        pltpu.VMEM((CB, D), jnp.float32),
        pltpu.VMEM((CB, D), jnp.float32),
        pltpu.SemaphoreType.DMA,
        pltpu.SemaphoreType.DMA,
        pltpu.SemaphoreType.DMA,
        pltpu.SemaphoreType.DMA,
        pltpu.VMEM_SHARED((NPAD, D), jnp.float32),
    ],
    compiler_params=pltpu.CompilerParams(needs_layout_passes=False),
)
def _sc_aggregate(h_dd, h_dt, h_tt, att_dd, att_dt, att_tt,
                  src2_dd, src2_dt, src2_tt, dst2_dd, dst2_dt, dst2_tt,
                  out_hbm,
                  src2_v, dst_r, att_r, rows_a, rows_b,
                  gsem_a, gsem_b, ssem_a, ssem_b, accum):
    cid = lax.axis_index("c")
    sid = lax.axis_index("s")
    wid = cid * NS + sid
    base = pl.multiple_of(wid * EPW, 8)
    h_hbms = (h_dd, h_dt, h_tt)
    att_hbms = (att_dd, att_dt, att_tt)
    src_hbms = (src2_dd, src2_dt, src2_tt)
    dst_hbms = (dst2_dd, dst2_dt, dst2_tt)
    rows = (rows_a, rows_b)
    gsems = (gsem_a, gsem_b)
    ssems = (ssem_a, ssem_b)

    for i in range(CB):
        for u in range(D // 16):
            rows_a[i, pl.ds(u * 16, 16)] = jnp.zeros((16,), jnp.float32)
    for q in range(RPT // CB):
        st = pl.multiple_of(sid * RPT + q * CB, 8)
        pltpu.sync_copy(rows_a, accum.at[pl.ds(st, CB)])
    plsc.subcore_barrier()

    for r in range(3):
        pltpu.sync_copy(src_hbms[r].at[wid], src2_v)

        def prefetch(ch, b, r=r):
            cbase = pl.multiple_of(base + ch * CB, 8)
            pltpu.async_copy(att_hbms[r].at[pl.ds(cbase, CB)],
                             att_r.at[b], gsems[b])
            pltpu.async_copy(dst_hbms[r].at[wid * NCH + ch],
                             dst_r.at[b], gsems[b])
            pltpu.async_copy(h_hbms[r].at[src2_v.at[ch]], rows[b], gsems[b])

        def process(ch, b, r=r):
            cbase = pl.multiple_of(base + ch * CB, 8)
            pltpu.make_async_copy(att_hbms[r].at[pl.ds(cbase, CB)],
                                  att_r.at[b], gsems[b]).wait()
            pltpu.make_async_copy(dst_hbms[r].at[wid * NCH + ch],
                                  dst_r.at[b], gsems[b]).wait()
            pltpu.make_async_copy(h_hbms[r].at[src2_v.at[ch]],
                                  rows[b], gsems[b]).wait()

            def scale_body(g, c2):
                av16 = att_r[b, pl.ds(g * 16, 16)]
                for j in range(16):
                    av = jnp.full((16,), av16[j], jnp.float32)
                    i = g * 16 + j
                    for u in range(D // 16):
                        sl = pl.ds(u * 16, 16)
                        rows[b][i, sl] = rows[b][i, sl] * av
                return c2

            lax.fori_loop(0, LPC, scale_body, 0)
            return pltpu.async_copy(rows[b], accum.at[dst_r.at[b]],
                                    ssems[b], add=True)

        prefetch(0, 0)
        prefetch(1, 1)

        def pair_body(p, c):
            sa = process(2 * p, 0)
            sb = process(2 * p + 1, 1)
            sa.wait()
            prefetch(2 * p + 2, 0)
            sb.wait()

            @pl.when(p < (NCH - 1) // 2 - 1)
            def _():
                prefetch(2 * p + 3, 1)

            return c

        lax.fori_loop(0, (NCH - 1) // 2, pair_body, 0)
        process(NCH - 1, 0).wait()

    plsc.subcore_barrier()
    for q in range(RPT // CB):
        st = pl.multiple_of(sid * RPT + q * CB, 8)
        pltpu.sync_copy(accum.at[pl.ds(st, CB)],
                        out_hbm.at[cid, pl.ds(st, CB)])


def _combine_body(p0_ref, p1_ref, o_ref):
    o_ref[...] = (p0_ref[...] + p1_ref[...]) * jnp.float32(1.0 / 3.0)


def _combine(p0, p1):
    blk = 1024
    spec = pl.BlockSpec((blk, D), lambda i: (i, 0))
    return pl.pallas_call(
        _combine_body,
        grid=(NPAD // blk,),
        in_specs=[spec, spec],
        out_specs=spec,
        out_shape=jax.ShapeDtypeStruct((NPAD, D), jnp.float32),
    )(p0, p1)


def kernel(all_nodes, edge_index_dd, edge_index_dt, edge_index_tt, edge_attr_dd,
           W_dd, a_src_dd, a_dst_dd, a_edge_dd,
           W_dt, a_src_dt, a_dst_dt,
           W_tt, a_src_tt, a_dst_tt):
    x = jnp.concatenate(
        [all_nodes, jnp.zeros((NPAD - N, D), jnp.float32)], axis=0)
    a6 = [a.reshape(1, D) for a in
          (a_src_dd, a_dst_dd, a_src_dt, a_dst_dt, a_src_tt, a_dst_tt)]
    (h_dd, h_dt, h_tt,
     s2_dd, t2_dd, s2_dt, t2_dt, s2_tt, t2_tt) = _project(
        x, W_dd, W_dt, W_tt, a6)
    s_dd, t_dd = s2_dd.reshape(NPAD), t2_dd.reshape(NPAD)
    s_dt, t_dt = s2_dt.reshape(NPAD), t2_dt.reshape(NPAD)
    s_tt, t_tt = s2_tt.reshape(NPAD), t2_tt.reshape(NPAD)

    ea2 = edge_attr_dd.reshape(E // 8, 8 * DE)
    rows = ((jnp.arange(8) * DE)[:, None] + jnp.arange(DE)[None, :]).reshape(-1)
    cols = jnp.repeat(jnp.arange(8), DE)
    B = jnp.zeros((8 * DE, 128), jnp.float32).at[rows, cols].set(
        jnp.tile(a_edge_dd, 8))
    eatt = _edge_term(ea2, B)[:, :8].reshape(E)

    src_dd, dst_dd = edge_index_dd[0], edge_index_dd[1]
    src_dt, dst_dt = edge_index_dt[0], edge_index_dt[1]
    src_tt, dst_tt = edge_index_tt[0], edge_index_tt[1]
    d2 = lambda a: a.reshape(NW, NCH, CB)

    (ex_dd, ex_dt, ex_tt, denp_dd, denp_dt, denp_tt) = _sc_edge_logits(
        s_dd, t_dd, s_dt, t_dt, s_tt, t_tt,
        src_dd, src_dt, src_tt, d2(dst_dd), d2(dst_dt), d2(dst_tt), eatt)

    den_dd, den_dt, den_tt = [d.reshape(NPAD) for d in
                              _den_combine((denp_dd, denp_dt, denp_tt))]

    att_dd, att_dt, att_tt = _sc_attn(
        ex_dd, ex_dt, ex_tt, d2(dst_dd), d2(dst_dt), d2(dst_tt),
        den_dd, den_dt, den_tt)

    dflat = lambda a: a.reshape(NW * NCH, CB)
    out_parts = _sc_aggregate(
        h_dd, h_dt, h_tt, att_dd, att_dt, att_tt,
        d2(src_dd), d2(src_dt), d2(src_tt),
        dflat(dst_dd), dflat(dst_dt), dflat(dst_tt))

    out = _combine(out_parts[0], out_parts[1])
    return out[:N]
